# baseline (speedup 1.0000x reference)
import functools

import jax
import jax.numpy as jnp
from jax import lax
from jax.experimental import pallas as pl
from jax.experimental.pallas import tpu as pltpu
from jax.experimental.pallas import tpu_sc as plsc

_NC, _NS, _L = 2, 16, 16
_NW = _NC * _NS
_ROWS = 16384
_COLS = 4096
_K = 128

_SC_ROWS = 8192
_TC_ROWS = _ROWS - _SC_ROWS

_RPW = _SC_ROWS // _NW
_RB = 8
_NBLK = _RPW // _RB
_NCHUNK = _COLS // _L

_BR = 1024
_BC = 512


@functools.partial(
    pl.kernel,
    out_type=jax.ShapeDtypeStruct((_SC_ROWS, _COLS), jnp.float32),
    name="sc_coeff_expand",
    compiler_params=pltpu.CompilerParams(needs_layout_passes=False),
    mesh=plsc.VectorSubcoreMesh(core_axis_name="c", subcore_axis_name="s"),
    scratch_types=[
        pltpu.VMEM((_COLS,), jnp.int32),
        pltpu.VMEM((_RB * _K,), jnp.float32),
        pltpu.VMEM((_RB * _K,), jnp.float32),
        pltpu.VMEM((_RB, _COLS), jnp.float32),
        pltpu.VMEM((_RB, _COLS), jnp.float32),
        pltpu.SemaphoreType.DMA,
        pltpu.SemaphoreType.DMA,
        pltpu.SemaphoreType.DMA,
        pltpu.SemaphoreType.DMA,
    ],
)
def _sc_expand(x_hbm, mask_hbm, out_hbm, mask_v, x0, x1, o0, o1,
               sx0, sx1, so0, so1):
    wid = lax.axis_index("s") * _NC + lax.axis_index("c")
    base = wid * _RPW
    pltpu.sync_copy(mask_hbm, mask_v)

    xb = (x0, x1)
    ob = (o0, o1)
    sx = (sx0, sx1)
    so = (so0, so1)

    def x_src(b):
        return x_hbm.at[pl.ds((base + b * _RB) * _K, _RB * _K)]

    def out_dst(b):
        return out_hbm.at[pl.ds(base + b * _RB, _RB)]

    pltpu.async_copy(x_src(0), x0, sx0)
    pltpu.async_copy(x_src(1), x1, sx1)

    def step(t, carry):
        for p in range(2):
            b = 2 * t + p
            x_ref, out_ref = xb[p], ob[p]

            @pl.when(b >= 2)
            def _():
                pltpu.make_async_copy(out_ref, out_dst(b - 2), so[p]).wait()

            pltpu.make_async_copy(x_src(b), x_ref, sx[p]).wait()

            @plsc.parallel_loop(0, _NCHUNK, unroll=2)
            def _(j):
                m = mask_v[pl.ds(j * _L, _L)]
                for r in range(_RB):
                    out_ref[r, pl.ds(j * _L, _L)] = plsc.load_gather(
                        x_ref, [m + (r * _K)])

            @pl.when(b + 2 < _NBLK)
            def _():
                pltpu.async_copy(x_src(b + 2), x_ref, sx[p])

            pltpu.async_copy(out_ref, out_dst(b), so[p])
        return carry

    lax.fori_loop(0, _NBLK // 2, step, 0)
    pltpu.make_async_copy(o0, out_dst(_NBLK - 2), so0).wait()
    pltpu.make_async_copy(o1, out_dst(_NBLK - 1), so1).wait()


def _tc_body(mask_ref, x_ref, out_ref):
    m = mask_ref[0, 0, :]
    iota = lax.broadcasted_iota(jnp.int32, (_K, _BC), 0)
    onehot = (iota == m[None, :]).astype(jnp.float32)
    out_ref[...] = jnp.dot(x_ref[...], onehot,
                           preferred_element_type=jnp.float32)


def _tc_expand(x, mask):
    mask3 = mask.reshape(_COLS // _BC, 1, _BC)
    return pl.pallas_call(
        _tc_body,
        grid=(_TC_ROWS // _BR, _COLS // _BC),
        in_specs=[
            pl.BlockSpec((1, 1, _BC), lambda i, j: (j, 0, 0)),
            pl.BlockSpec((_BR, _K), lambda i, j: (i, 0)),
        ],
        out_specs=pl.BlockSpec((_BR, _BC), lambda i, j: (i, j)),
        out_shape=jax.ShapeDtypeStruct((_TC_ROWS, _COLS), jnp.float32),
    )(mask3, x)


def kernel(x, mask):
    out_sc = _sc_expand(x[_TC_ROWS:].reshape(-1), mask)
    out_tc = _tc_expand(x[:_TC_ROWS], mask)
    return jnp.concatenate([out_tc, out_sc], axis=0)


# speedup vs baseline: 3.3407x; 3.3407x over previous
import jax
import jax.numpy as jnp
from jax import lax
from jax.experimental import pallas as pl

_BR = 1024
_BC = 4096
_N_ROWS = 16384
_N_COLS = 4096
_K = 128


def _tc_body(mask_ref, x_ref, out_ref):
    m = mask_ref[0, :]
    iota = lax.broadcasted_iota(jnp.int32, (_K, _BC), 0)
    onehot = (iota == m[None, :]).astype(jnp.float32)
    out_ref[...] = jnp.dot(x_ref[...], onehot,
                           preferred_element_type=jnp.float32)


def kernel(x, mask):
    return pl.pallas_call(
        _tc_body,
        grid=(_N_ROWS // _BR,),
        in_specs=[
            pl.BlockSpec((1, _BC), lambda i: (0, 0)),
            pl.BlockSpec((_BR, _K), lambda i: (i, 0)),
        ],
        out_specs=pl.BlockSpec((_BR, _BC), lambda i: (i, 0)),
        out_shape=jax.ShapeDtypeStruct((_N_ROWS, _N_COLS), jnp.float32),
    )(mask.reshape(1, _N_COLS), x)


# speedup vs baseline: 3.3553x; 1.0044x over previous
import jax
import jax.numpy as jnp
from jax import lax
from jax.experimental import pallas as pl

_BR = 512
_BC = 4096
_N_ROWS = 16384
_N_COLS = 4096
_K = 128


def _tc_body(mask_ref, x_ref, out_ref):
    m = mask_ref[0, :]
    iota = lax.broadcasted_iota(jnp.int32, (_K, _BC), 0)
    onehot = (iota == m[None, :]).astype(jnp.float32)
    out_ref[...] = jnp.dot(x_ref[...], onehot,
                           preferred_element_type=jnp.float32)


def kernel(x, mask):
    return pl.pallas_call(
        _tc_body,
        grid=(_N_ROWS // _BR,),
        in_specs=[
            pl.BlockSpec((1, _BC), lambda i: (0, 0)),
            pl.BlockSpec((_BR, _K), lambda i: (i, 0)),
        ],
        out_specs=pl.BlockSpec((_BR, _BC), lambda i: (i, 0)),
        out_shape=jax.ShapeDtypeStruct((_N_ROWS, _N_COLS), jnp.float32),
    )(mask.reshape(1, _N_COLS), x)
